# baseline (speedup 1.0000x reference)
import functools

import jax
import jax.numpy as jnp
from jax import lax
from jax.experimental import pallas as pl
from jax.experimental.pallas import tpu as pltpu
from jax.experimental.pallas import tpu_sc as plsc

_NUM_CORES = 2
_LANES = 16
_LANE_BLOCK = 128
_ROWS_PER_WORKER = 8


@functools.lru_cache(maxsize=None)
def _make_gather(B, V, D):
    n_active = B // _ROWS_PER_WORKER
    mesh = plsc.VectorSubcoreMesh(core_axis_name="c", subcore_axis_name="s")

    @functools.partial(
        pl.kernel,
        out_type=jax.ShapeDtypeStruct((B, D), jnp.float32),
        mesh=mesh,
        scratch_types=[
            pltpu.VMEM((_LANES,), jnp.int32),
            pltpu.VMEM((_ROWS_PER_WORKER, D, _LANE_BLOCK), jnp.float32),
            pltpu.VMEM((_ROWS_PER_WORKER, D), jnp.float32),
            pltpu.SemaphoreType.DMA,
            pltpu.SemaphoreType.DMA,
        ],
        compiler_params=pltpu.CompilerParams(needs_layout_passes=False),
    )
    def gather(tablet_hbm, idx_hbm, out_hbm, idx_v, win_v, out_v, sem_a,
               sem_b):
        wid = lax.axis_index("s") * _NUM_CORES + lax.axis_index("c")

        @pl.when(wid < n_active)
        def _():
            base = wid * _ROWS_PER_WORKER
            pltpu.sync_copy(idx_hbm.at[pl.ds(base, _ROWS_PER_WORKER)],
                            idx_v.at[pl.ds(0, _ROWS_PER_WORKER)])
            v = idx_v[...]
            blk = lax.bitwise_and(v, -_LANE_BLOCK)
            lane = lax.bitwise_and(v, _LANE_BLOCK - 1)
            halfn = _ROWS_PER_WORKER // 2
            copies = []
            for k in range(_ROWS_PER_WORKER):
                start = pl.multiple_of(blk[k], _LANE_BLOCK)
                copies.append(
                    pltpu.async_copy(
                        tablet_hbm.at[:, pl.ds(start, _LANE_BLOCK)],
                        win_v.at[k], sem_a if k < halfn else sem_b))
            rows_b = [lax.iota(jnp.int32, _LANES) + (b * _LANES)
                      for b in range(D // _LANES)]

            def extract(k):
                lane_k = jnp.broadcast_to(lane[k], (_LANES,))
                for b in range(D // _LANES):
                    seg = plsc.load_gather(win_v.at[k], [rows_b[b], lane_k])
                    out_v[k, pl.ds(b * _LANES, _LANES)] = seg

            for k in range(halfn):
                copies[k].wait()
            for k in range(halfn):
                extract(k)
            for k in range(halfn, _ROWS_PER_WORKER):
                copies[k].wait()
            for k in range(halfn, _ROWS_PER_WORKER):
                extract(k)
            pltpu.sync_copy(out_v, out_hbm.at[pl.ds(base, _ROWS_PER_WORKER)])

    return gather


def kernel(layer_values, ordinals):
    V, D = layer_values.shape
    (B,) = ordinals.shape
    return _make_gather(B, V, D)(layer_values.T, ordinals.astype(jnp.int32))


# speedup vs baseline: 1.0085x; 1.0085x over previous
import functools

import jax
import jax.numpy as jnp
from jax import lax
from jax.experimental import pallas as pl
from jax.experimental.pallas import tpu as pltpu
from jax.experimental.pallas import tpu_sc as plsc

_NUM_CORES = 2
_LANES = 16
_LANE_BLOCK = 128
_ROWS_PER_WORKER = 8


@functools.lru_cache(maxsize=None)
def _make_gather(B, V, D):
    n_active = B // _ROWS_PER_WORKER
    mesh = plsc.VectorSubcoreMesh(core_axis_name="c", subcore_axis_name="s")

    @functools.partial(
        pl.kernel,
        out_type=jax.ShapeDtypeStruct((B, D), jnp.float32),
        mesh=mesh,
        scratch_types=[
            pltpu.VMEM((_LANES,), jnp.int32),
            pltpu.VMEM((_ROWS_PER_WORKER, D, _LANE_BLOCK), jnp.float32),
            pltpu.VMEM((_ROWS_PER_WORKER, D), jnp.float32),
            pltpu.SemaphoreType.DMA,
        ],
        compiler_params=pltpu.CompilerParams(needs_layout_passes=False),
    )
    def gather(tablet_hbm, idx_hbm, out_hbm, idx_v, win_v, out_v, sem):
        wid = lax.axis_index("s") * _NUM_CORES + lax.axis_index("c")

        @pl.when(wid < n_active)
        def _():
            base = wid * _ROWS_PER_WORKER
            pltpu.sync_copy(idx_hbm.at[pl.ds(base, _ROWS_PER_WORKER)],
                            idx_v.at[pl.ds(0, _ROWS_PER_WORKER)])
            v = idx_v[...]
            blk = lax.bitwise_and(v, -_LANE_BLOCK)
            lane = lax.bitwise_and(v, _LANE_BLOCK - 1)
            copies = []
            for k in range(_ROWS_PER_WORKER):
                start = pl.multiple_of(blk[k], _LANE_BLOCK)
                copies.append(
                    pltpu.async_copy(
                        tablet_hbm.at[:, pl.ds(start, _LANE_BLOCK)],
                        win_v.at[k], sem))
            for c in copies:
                c.wait()
            for k in range(_ROWS_PER_WORKER):
                lane_k = jnp.broadcast_to(lane[k], (_LANES,))
                for b in range(D // _LANES):
                    rows = lax.iota(jnp.int32, _LANES) + (b * _LANES)
                    seg = plsc.load_gather(win_v.at[k], [rows, lane_k])
                    out_v[k, pl.ds(b * _LANES, _LANES)] = seg
            pltpu.sync_copy(out_v, out_hbm.at[pl.ds(base, _ROWS_PER_WORKER)])

    return gather


def kernel(layer_values, ordinals):
    V, D = layer_values.shape
    (B,) = ordinals.shape
    return _make_gather(B, V, D)(layer_values.T, ordinals.astype(jnp.int32))
